# baseline (speedup 1.0000x reference)
import jax
import jax.numpy as jnp
import numpy as np
from jax.experimental import pallas as pl
from jax.experimental.pallas import tpu as pltpu

B = 128
N = 100000
BN = 2048
K = (N + BN - 1) // BN
NEG = -1e30
LOGMIN = float(np.log(np.float32(1e-30)))


def _stats_kernel(scores_ref, av_ref, act_ref,
                  m_ref, t_ref, u_ref, z_ref, sa_ref, aa_ref):
    k = pl.program_id(0)

    @pl.when(k == 0)
    def _init():
        m_ref[...] = jnp.full((B, 1), NEG, jnp.float32)
        t_ref[...] = jnp.zeros((B, 1), jnp.float32)
        u_ref[...] = jnp.zeros((B, 1), jnp.float32)
        z_ref[...] = jnp.zeros((B, 1), jnp.float32)
        sa_ref[...] = jnp.zeros((B, 1), jnp.float32)
        aa_ref[...] = jnp.zeros((B, 1), jnp.float32)

    s = scores_ref[...]
    a = av_ref[...]
    col = jax.lax.broadcasted_iota(jnp.int32, (B, BN), 1) + k * BN
    valid = col < N
    am = (a > 0) & valid

    m_old = m_ref[...]
    bm = jnp.max(jnp.where(am, s, NEG), axis=1, keepdims=True)
    m_new = jnp.maximum(m_old, bm)
    scale = jnp.exp(m_old - m_new)
    e = jnp.where(am, jnp.exp(s - m_new), 0.0)
    bt = jnp.sum(e, axis=1, keepdims=True)
    bu = jnp.sum(jnp.where(am, e * (s - m_new), 0.0), axis=1, keepdims=True)
    t_old = t_ref[...]
    u_old = u_ref[...]
    m_ref[...] = m_new
    t_ref[...] = t_old * scale + bt
    u_ref[...] = (u_old - (m_new - m_old) * t_old) * scale + bu
    z_ref[...] = z_ref[...] + jnp.sum(
        jnp.where(valid & (a <= 0), 1.0, 0.0), axis=1, keepdims=True)

    amask = col == act_ref[...]
    sa_ref[...] = sa_ref[...] + jnp.sum(
        jnp.where(amask, s, 0.0), axis=1, keepdims=True)
    aa_ref[...] = aa_ref[...] + jnp.sum(
        jnp.where(amask, a.astype(jnp.float32), 0.0), axis=1, keepdims=True)


def _finalize_kernel(scores_ref, av_ref,
                     m_ref, t_ref, u_ref, z_ref, sa_ref, aa_ref,
                     probs_ref, lp_ref, ent_ref):
    k = pl.program_id(0)
    s = scores_ref[...]
    a = av_ref[...]
    col = jax.lax.broadcasted_iota(jnp.int32, (B, BN), 1) + k * BN
    am = (a > 0) & (col < N)
    m = m_ref[...]
    t = t_ref[...]
    z = z_ref[...]
    D = t + 1e-13 * (t + z)
    invD = 1.0 / D
    e = jnp.where(am, jnp.exp(s - m), 0.0)
    probs_ref[...] = e * invD

    @pl.when(k == 0)
    def _heads():
        u = u_ref[...]
        sa = sa_ref[...]
        aa = aa_ref[...]
        logD = jnp.log(D)
        ent_ref[...] = (logD * t - u) * invD
        lp_ref[...] = jnp.where(
            aa > 0, jnp.maximum(sa - m - logD, LOGMIN), LOGMIN)


def _row_stat_spec():
    return pl.BlockSpec((B, 1), lambda k: (0, 0))


def kernel(scores, available, action):
    act2 = action.reshape(B, 1).astype(jnp.int32)
    stat_shape = jax.ShapeDtypeStruct((B, 1), jnp.float32)

    stats = pl.pallas_call(
        _stats_kernel,
        grid=(K,),
        in_specs=[
            pl.BlockSpec((B, BN), lambda k: (0, k)),
            pl.BlockSpec((B, BN), lambda k: (0, k)),
            pl.BlockSpec((B, 1), lambda k: (0, 0)),
        ],
        out_specs=[_row_stat_spec() for _ in range(6)],
        out_shape=[stat_shape for _ in range(6)],
    )(scores, available, act2)

    probs, lp, ent = pl.pallas_call(
        _finalize_kernel,
        grid=(K,),
        in_specs=[
            pl.BlockSpec((B, BN), lambda k: (0, k)),
            pl.BlockSpec((B, BN), lambda k: (0, k)),
        ] + [_row_stat_spec() for _ in range(6)],
        out_specs=[
            pl.BlockSpec((B, BN), lambda k: (0, k)),
            _row_stat_spec(),
            _row_stat_spec(),
        ],
        out_shape=[
            jax.ShapeDtypeStruct((B, N), jnp.float32),
            stat_shape,
            stat_shape,
        ],
    )(scores, available, *stats)

    return lp.reshape(B), ent.reshape(B), probs
